# Initial kernel scaffold; baseline (speedup 1.0000x reference)
import jax
import jax.numpy as jnp
from jax.experimental import pallas as pl


def kernel(item_seq, tables, fusion_weights):
    raise NotImplementedError("write your pallas kernel here")


# speedup vs baseline: 3.6658x; 3.6658x over previous
import functools

import jax
import jax.numpy as jnp
from jax import lax
from jax.experimental import pallas as pl
from jax.experimental.pallas import tpu as pltpu
from jax.experimental.pallas import tpu_sc as plsc

_B, _S, _D = 4096, 200, 64
_N = _B * _S
_TBL = 1024

_info = plsc.get_sparse_core_info()
_NC, _NS, _L = _info.num_cores, _info.num_subcores, _info.num_lanes
_NW = _NC * _NS
_PER_W = _N // _NW
_CHUNK = 128
_NCHUNK = _PER_W // _CHUNK

_mesh = plsc.VectorSubcoreMesh(core_axis_name="c", subcore_axis_name="s")


@functools.partial(
    pl.kernel,
    mesh=_mesh,
    out_type=jax.ShapeDtypeStruct((_N, _D), jnp.float32),
    compiler_params=pltpu.CompilerParams(use_tc_tiling_on_sc=False),
    scratch_types=[
        pltpu.VMEM((2, 16), jnp.float32),
        pltpu.VMEM((_CHUNK,), jnp.int32),
        pltpu.VMEM((_CHUNK,), jnp.int32),
        pltpu.VMEM((_CHUNK,), jnp.int32),
        pltpu.VMEM((_CHUNK, _D), jnp.float32),
        pltpu.VMEM((_CHUNK, _D), jnp.float32),
        pltpu.VMEM((_CHUNK, _D), jnp.float32),
        pltpu.SemaphoreType.DMA,
        pltpu.SemaphoreType.DMA,
    ],
)
def _sc_fused_lookup(items_hbm, table_hbm, w_hbm, out_hbm,
                     w_v, item_v, idx0_v, idx1_v, rows0_v, rows1_v, out_v,
                     sem0, sem1):
    wid = lax.axis_index("s") * _NC + lax.axis_index("c")
    base = wid * _PER_W

    pltpu.sync_copy(w_hbm, w_v)
    e0 = jnp.exp(w_v[0, :])
    e1 = jnp.exp(w_v[1, :])
    w0 = e0 / (e0 + e1)
    w1 = e1 / (e0 + e1)

    def chunk_body(g, carry):
        cbase = base + g * _CHUNK
        pltpu.sync_copy(items_hbm.at[pl.ds(cbase, _CHUNK)], item_v)
        for j in range(_CHUNK // _L):
            v = item_v[pl.ds(j * _L, _L)]
            idx0_v[pl.ds(j * _L, _L)] = v & (_TBL - 1)
            idx1_v[pl.ds(j * _L, _L)] = (v >> 10) + _TBL
        cp0 = pltpu.async_copy(table_hbm.at[idx0_v], rows0_v, sem0)
        cp1 = pltpu.async_copy(table_hbm.at[idx1_v], rows1_v, sem1)
        cp0.wait()
        cp1.wait()

        def row_body(r, carry2):
            a = rows0_v.at[r]
            b = rows1_v.at[r]
            o = out_v.at[r]
            for cstart in range(0, _D, _L):
                sl = pl.ds(cstart, _L)
                o[sl] = a[sl] * w0 + b[sl] * w1
            return carry2

        lax.fori_loop(0, _CHUNK, row_body, 0, unroll=2)
        pltpu.sync_copy(out_v, out_hbm.at[pl.ds(cbase, _CHUNK)])
        return carry

    lax.fori_loop(0, _NCHUNK, chunk_body, 0)


def kernel(item_seq, tables, fusion_weights):
    items_flat = item_seq.reshape(_N)
    table2d = tables.reshape(2 * _TBL, _D)
    w_pad = jnp.broadcast_to(fusion_weights.reshape(2, 1), (2, 16))
    out = _sc_fused_lookup(items_flat, table2d, w_pad)
    return out.reshape(_B, _S, _D)


# speedup vs baseline: 6.2763x; 1.7121x over previous
import functools

import jax
import jax.numpy as jnp
from jax import lax
from jax.experimental import pallas as pl
from jax.experimental.pallas import tpu as pltpu
from jax.experimental.pallas import tpu_sc as plsc

_B, _S, _D = 4096, 200, 64
_N = _B * _S
_TBL = 1024

_info = plsc.get_sparse_core_info()
_NC, _NS, _L = _info.num_cores, _info.num_subcores, _info.num_lanes
_NW = _NC * _NS
_PER_W = _N // _NW
_CHUNK = 128
_NCHUNK = _PER_W // _CHUNK
_SLOTS = 4
_OUTER = _NCHUNK // _SLOTS

_mesh = plsc.VectorSubcoreMesh(core_axis_name="c", subcore_axis_name="s")


@functools.partial(
    pl.kernel,
    mesh=_mesh,
    out_type=jax.ShapeDtypeStruct((_N, _D), jnp.float32),
    compiler_params=pltpu.CompilerParams(use_tc_tiling_on_sc=False),
    scratch_types=[
        pltpu.VMEM((2, 16), jnp.float32),
        pltpu.VMEM((_NCHUNK, _CHUNK), jnp.int32),
        pltpu.VMEM((_SLOTS, _CHUNK), jnp.int32),
        pltpu.VMEM((_SLOTS, _CHUNK), jnp.int32),
        pltpu.VMEM((_SLOTS, _CHUNK, _D), jnp.float32),
        pltpu.VMEM((_SLOTS, _CHUNK, _D), jnp.float32),
        pltpu.VMEM((_SLOTS, _CHUNK, _D), jnp.float32),
        pltpu.SemaphoreType.DMA,
        pltpu.SemaphoreType.DMA,
        pltpu.SemaphoreType.DMA,
        pltpu.SemaphoreType.DMA,
        pltpu.SemaphoreType.DMA,
        pltpu.SemaphoreType.DMA,
        pltpu.SemaphoreType.DMA,
        pltpu.SemaphoreType.DMA,
    ],
)
def _sc_fused_lookup(items_hbm, table_hbm, w_hbm, out_hbm,
                     w_v, item_all, idx0_v, idx1_v, rowsA, rowsB, out_v,
                     gs0, gs1, gs2, gs3, ow0, ow1, ow2, ow3):
    gs = (gs0, gs1, gs2, gs3)
    ow = (ow0, ow1, ow2, ow3)
    wid = lax.axis_index("s") * _NC + lax.axis_index("c")
    row_base = wid * _PER_W

    pltpu.sync_copy(w_hbm, w_v)
    e0 = jnp.exp(w_v[0, :])
    e1 = jnp.exp(w_v[1, :])
    w0 = e0 / (e0 + e1)
    w1 = e1 / (e0 + e1)

    pltpu.sync_copy(items_hbm.at[pl.ds(wid * _NCHUNK, _NCHUNK)], item_all)

    def compute_idx(chunk, s):
        src = item_all.at[chunk]
        d0 = idx0_v.at[s]
        d1 = idx1_v.at[s]
        for j in range(_CHUNK // _L):
            sl = pl.ds(j * _L, _L)
            v = src[sl]
            d0[sl] = v & (_TBL - 1)
            d1[sl] = (v >> 10) + _TBL

    def issue_gathers(s):
        pltpu.async_copy(table_hbm.at[idx0_v.at[s]], rowsA.at[s], gs[s])
        pltpu.async_copy(table_hbm.at[idx1_v.at[s]], rowsB.at[s], gs[s])

    def wait_gathers(s):
        pltpu.make_async_copy(table_hbm.at[idx0_v.at[s]], rowsA.at[s], gs[s]).wait()
        pltpu.make_async_copy(table_hbm.at[idx1_v.at[s]], rowsB.at[s], gs[s]).wait()

    def wait_out(s):
        pltpu.make_async_copy(
            out_v.at[s], out_hbm.at[pl.ds(row_base, _CHUNK)], ow[s]).wait()

    for s in range(_SLOTS):
        compute_idx(s, s)
        issue_gathers(s)

    def outer(i, carry):
        for s in range(_SLOTS):
            c = i * _SLOTS + s
            wait_gathers(s)

            @pl.when(i > 0)
            def _():
                wait_out(s)

            def row_body(r, carry2):
                a = rowsA.at[s].at[r]
                b = rowsB.at[s].at[r]
                o = out_v.at[s].at[r]
                for cstart in range(0, _D, _L):
                    sl = pl.ds(cstart, _L)
                    o[sl] = a[sl] * w0 + b[sl] * w1
                return carry2

            lax.fori_loop(0, _CHUNK, row_body, 0, unroll=4)

            pltpu.async_copy(
                out_v.at[s], out_hbm.at[pl.ds(row_base + c * _CHUNK, _CHUNK)], ow[s])

            @pl.when(i < _OUTER - 1)
            def _():
                compute_idx(c + _SLOTS, s)
                issue_gathers(s)

        return carry

    lax.fori_loop(0, _OUTER, outer, 0)

    for s in range(_SLOTS):
        wait_out(s)


def kernel(item_seq, tables, fusion_weights):
    items_2d = item_seq.reshape(_N // _CHUNK, _CHUNK)
    table2d = tables.reshape(2 * _TBL, _D)
    w_pad = jnp.broadcast_to(fusion_weights.reshape(2, 1), (2, 16))
    out = _sc_fused_lookup(items_2d, table2d, w_pad)
    return out.reshape(_B, _S, _D)
